# Initial kernel scaffold; baseline (speedup 1.0000x reference)
import jax
import jax.numpy as jnp
from jax.experimental import pallas as pl


def kernel(vertices, feature_map):
    raise NotImplementedError("write your pallas kernel here")


# speedup vs baseline: 8.6112x; 8.6112x over previous
import functools

import jax
import jax.numpy as jnp
from jax import lax
from jax.experimental import pallas as pl
from jax.experimental.pallas import tpu as pltpu
from jax.experimental.pallas import tpu_sc as plsc

_POOLING_RATE = 4
_NEIGHBOR_NUM = 4
_ANCHOR = 12
_ROW = 16


def _knn_topk_tc(vertices, queries):
    bs, V, _ = vertices.shape
    Q = queries.shape[2]
    QB = 256

    def body(v_ref, q_ref, o_ref):
        wx = v_ref[0, :, 0:1]
        wy = v_ref[0, :, 1:2]
        wz = v_ref[0, :, 2:3]
        qx = q_ref[0, 0:1, :]
        qy = q_ref[0, 1:2, :]
        qz = q_ref[0, 2:3, :]
        wn = wx * wx + wy * wy + wz * wz
        qn = qx * qx + qy * qy + qz * qz
        wxb = wx.astype(jnp.bfloat16).astype(jnp.float32)
        wyb = wy.astype(jnp.bfloat16).astype(jnp.float32)
        wzb = wz.astype(jnp.bfloat16).astype(jnp.float32)
        qxb = qx.astype(jnp.bfloat16).astype(jnp.float32)
        qyb = qy.astype(jnp.bfloat16).astype(jnp.float32)
        qzb = qz.astype(jnp.bfloat16).astype(jnp.float32)
        inner = (wxb * qxb + wyb * qyb) + wzb * qzb
        dist = (inner * (-2.0) + wn) + qn
        iota = lax.broadcasted_iota(jnp.int32, (V, QB), 0)
        big = jnp.int32(2 ** 30)
        for k in range(5):
            mval = jnp.min(dist, axis=0, keepdims=True)
            cand = jnp.where(dist == mval, iota, big)
            midx = jnp.min(cand, axis=0, keepdims=True)
            o_ref[0, k:k + 1, :] = midx
            if k < 4:
                dist = jnp.where(iota == midx, jnp.float32(jnp.inf), dist)

    return pl.pallas_call(
        body,
        grid=(bs, Q // QB),
        in_specs=[
            pl.BlockSpec((1, V, 3), lambda b, i: (b, 0, 0)),
            pl.BlockSpec((1, 3, QB), lambda b, i: (b, 0, i)),
        ],
        out_specs=pl.BlockSpec((1, 8, QB), lambda b, i: (b, 0, i)),
        out_shape=jax.ShapeDtypeStruct((bs, 8, Q), jnp.int32),
    )(vertices, queries)


def _gather_max_sc(nbr_flat, table, bs, C, V, P):
    info = plsc.get_sparse_core_info()
    NC, NS = info.num_cores, info.num_subcores
    NW = NC * NS
    BC = bs * C
    per_w = BC // NW
    H = 2
    QH = P // H
    RH = _NEIGHBOR_NUM * QH
    mesh = plsc.VectorSubcoreMesh(core_axis_name="c", subcore_axis_name="s")

    @functools.partial(
        pl.kernel,
        mesh=mesh,
        out_type=jax.ShapeDtypeStruct((BC, P, _ROW), jnp.float32),
        compiler_params=pltpu.CompilerParams(use_tc_tiling_on_sc=False),
        scratch_types=[
            pltpu.VMEM((_NEIGHBOR_NUM * P,), jnp.int32),
            pltpu.VMEM((RH // 128, 128), jnp.int32),
            pltpu.VMEM((RH, _ROW), jnp.float32),
            pltpu.VMEM((QH, _ROW), jnp.float32),
            pltpu.SemaphoreType.DMA,
        ],
    )
    def k(nbr_hbm, table_hbm, out_hbm, nbr_v, idx_v, buf, outb, sem):
        wid = lax.axis_index("s") * NC + lax.axis_index("c")
        b = wid // (NW // bs)
        pltpu.sync_copy(nbr_hbm.at[pl.ds(b * (_NEIGHBOR_NUM * P),
                                         _NEIGHBOR_NUM * P)], nbr_v)

        def chunk_body(t, carry):
            bc = wid * per_w + t // H
            h = t % H
            off = bc * V

            def build(i, c2):
                n = i // (QH // 16)
                q16 = (i % (QH // 16)) * 16
                v = nbr_v[pl.ds(n * P + h * QH + q16, 16)] + off
                idx_v[i // 8, pl.ds((i % 8) * 16, 16)] = v
                return c2

            lax.fori_loop(0, RH // 16, build, 0)

            cps = [
                pltpu.async_copy(table_hbm.at[idx_v.at[r]],
                                 buf.at[pl.ds(r * 128, 128)], sem)
                for r in range(RH // 128)
            ]
            for cp in cps:
                cp.wait()

            def mx(q, c2):
                m0 = jnp.maximum(buf[q, :], buf[QH + q, :])
                m1 = jnp.maximum(buf[2 * QH + q, :], buf[3 * QH + q, :])
                outb[q, :] = jnp.maximum(m0, m1)
                return c2

            lax.fori_loop(0, QH, mx, 0)
            pltpu.sync_copy(outb, out_hbm.at[bc, pl.ds(h * QH, QH)])
            return carry

        lax.fori_loop(0, per_w * H, chunk_body, 0)

    return k(nbr_flat, table)


def kernel(vertices, feature_map):
    bs, V, _ = vertices.shape
    C = feature_map.shape[1]
    P = V // _POOLING_RATE
    sample_idx = jax.random.permutation(jax.random.key(123), V)[:P]
    vertices_pool = vertices[:, sample_idx, :]
    queries = jnp.transpose(vertices_pool, (0, 2, 1))

    idx5 = _knn_topk_tc(vertices, queries)
    nbr = idx5[:, 1:1 + _NEIGHBOR_NUM, :]
    nbr_flat = nbr.reshape(bs * _NEIGHBOR_NUM * P)

    table = jnp.pad(feature_map, ((0, 0), (0, 0), (0, 0), (0, _ROW - _ANCHOR)))
    table = table.reshape(bs * C * V, _ROW)

    pooled = _gather_max_sc(nbr_flat, table, bs, C, V, P)
    feature_map_pool = pooled.reshape(bs, C, P, _ROW)[..., :_ANCHOR]
    return (vertices_pool, feature_map_pool)
